# Initial kernel scaffold; baseline (speedup 1.0000x reference)
import jax
import jax.numpy as jnp
from jax.experimental import pallas as pl


def kernel(x, edge_index, y, W1_self, W1_neigh, b1, W2_self, W2_neigh, b2, codebook, Wd1, bd1, Wd2, bd2):
    raise NotImplementedError("write your pallas kernel here")


# speedup vs baseline: 3.7131x; 3.7131x over previous
import functools

import jax
import jax.numpy as jnp
from jax import lax
from jax.experimental import pallas as pl
from jax.experimental.pallas import tpu as pltpu
from jax.experimental.pallas import tpu_sc as plsc

N = 50000
E = 800000
D_IN = 9
D_LAT = 128
K = 1024
BETA = 0.25

NC = 2
NS = 16
NA = 50048
TRASH = 50040
EROWS = 6400
EPAD = EROWS * 128 - E
EB = EROWS // 8

_mesh = plsc.VectorSubcoreMesh(core_axis_name="c", subcore_axis_name="s",
                               num_cores=NC, num_subcores=NS)


@functools.partial(
    pl.kernel,
    out_type=jax.ShapeDtypeStruct((2 * NA, 16), jnp.float32),
    mesh=_mesh,
    compiler_params=pltpu.CompilerParams(use_tc_tiling_on_sc=False),
    scratch_types=[
        pltpu.VMEM_SHARED((NA, 16), jnp.float32),
        pltpu.VMEM((1024,), jnp.int32),
        pltpu.VMEM((8, 128), jnp.int32),
        pltpu.VMEM((8, 128, 16), jnp.float32),
        pltpu.SemaphoreType.DMA,
        pltpu.SemaphoreType.DMA,
    ],
)
def _seg16(xpad_hbm, src_hbm, dst_hbm, zeros_hbm, out_hbm,
           acc, sbuf, dbuf, rows, gsem, ssem):
    c = lax.axis_index("c")
    t = lax.axis_index("s")
    pltpu.sync_copy(zeros_hbm.at[pl.ds(t * (NA // NS), NA // NS)],
                    acc.at[pl.ds(t * (NA // NS), NA // NS)])
    plsc.subcore_barrier()

    bb0 = c * (EB // 2) + t * (EB // 2 // NS)

    def body(b, carry):
        bb = bb0 + b
        pltpu.sync_copy(src_hbm.at[pl.ds(bb * 1024, 1024)], sbuf)
        pltpu.sync_copy(dst_hbm.at[bb], dbuf)
        gs = [pltpu.async_copy(xpad_hbm.at[sbuf.at[pl.ds(j * 128, 128)]],
                               rows.at[j], gsem)
              for j in range(8)]
        for g in gs:
            g.wait()
        ss = [pltpu.async_copy(rows.at[j], acc.at[dbuf.at[j]], ssem, add=True)
              for j in range(8)]
        for s in ss:
            s.wait()
        return carry

    lax.fori_loop(0, EB // 2 // NS, body, 0)
    plsc.subcore_barrier()
    pltpu.sync_copy(acc.at[pl.ds(t * (NA // NS), NA // NS)],
                    out_hbm.at[pl.ds(c * NA + t * (NA // NS), NA // NS)])


@functools.partial(
    pl.kernel,
    out_type=jax.ShapeDtypeStruct((8 * NA, 16), jnp.float32),
    mesh=_mesh,
    compiler_params=pltpu.CompilerParams(use_tc_tiling_on_sc=False),
    scratch_types=[
        pltpu.VMEM_SHARED((NA, 16), jnp.float32),
        pltpu.VMEM((1024,), jnp.int32),
        pltpu.VMEM((8, 128), jnp.int32),
        pltpu.VMEM((8, 128, 16), jnp.float32),
        pltpu.SemaphoreType.DMA,
        pltpu.SemaphoreType.DMA,
    ],
)
def _seg32(hcat_hbm, src_hbm, dst_hbm, zeros_hbm, out_hbm,
           acc, sbuf, dbuf, rows, gsem, ssem):
    c = lax.axis_index("c")
    t = lax.axis_index("s")
    for k in range(4):
        chunk = c * 4 + k
        off = chunk * N
        pltpu.sync_copy(zeros_hbm.at[pl.ds(t * (NA // NS), NA // NS)],
                        acc.at[pl.ds(t * (NA // NS), NA // NS)])
        plsc.subcore_barrier()

        bb0 = t * (EB // NS)

        def body(b, carry):
            bb = bb0 + b
            pltpu.sync_copy(src_hbm.at[pl.ds(bb * 1024, 1024)], sbuf)
            pltpu.sync_copy(dst_hbm.at[bb], dbuf)
            offv = jnp.full((16,), off, jnp.int32)
            for i in range(64):
                sl = pl.ds(i * 16, 16)
                sbuf[sl] = sbuf[sl] + offv
            gs = [pltpu.async_copy(hcat_hbm.at[sbuf.at[pl.ds(j * 128, 128)]],
                                   rows.at[j], gsem)
                  for j in range(8)]
            for g in gs:
                g.wait()
            ss = [pltpu.async_copy(rows.at[j], acc.at[dbuf.at[j]], ssem,
                                   add=True)
                  for j in range(8)]
            for s in ss:
                s.wait()
            return carry

        lax.fori_loop(0, EB // NS, body, 0)
        plsc.subcore_barrier()
        pltpu.sync_copy(acc.at[pl.ds(t * (NA // NS), NA // NS)],
                        out_hbm.at[pl.ds(chunk * NA + t * (NA // NS), NA // NS)])
        plsc.subcore_barrier()


_BN = 400


def _enc_body(xp_ref, p_ref, w1s_ref, w1n_ref, b1_ref, h4_ref, deg_ref):
    p = p_ref[0] + p_ref[1]
    deg = jnp.maximum(p[:, 9:10], 1.0)
    agg = p / deg
    hp = (jnp.dot(xp_ref[...], w1s_ref[...], preferred_element_type=jnp.float32)
          + jnp.dot(agg, w1n_ref[...], preferred_element_type=jnp.float32)
          + b1_ref[...])
    h = jnp.maximum(hp, 0.0)
    h4_ref[...] = jnp.stack([h[:, 16 * j:16 * j + 16] for j in range(8)])
    deg_ref[...] = deg


def _enc(xpad, parts, w1s, w1n, b1):
    return pl.pallas_call(
        _enc_body,
        grid=(N // _BN,),
        in_specs=[
            pl.BlockSpec((_BN, 16), lambda i: (i, 0)),
            pl.BlockSpec((2, _BN, 16), lambda i: (0, i, 0)),
            pl.BlockSpec((16, D_LAT), lambda i: (0, 0)),
            pl.BlockSpec((16, D_LAT), lambda i: (0, 0)),
            pl.BlockSpec((1, D_LAT), lambda i: (0, 0)),
        ],
        out_specs=[
            pl.BlockSpec((8, _BN, 16), lambda i: (0, i, 0)),
            pl.BlockSpec((_BN, 1), lambda i: (i, 0)),
        ],
        out_shape=[
            jax.ShapeDtypeStruct((8, N, 16), jnp.float32),
            jax.ShapeDtypeStruct((N, 1), jnp.float32),
        ],
    )(xpad, parts, w1s, w1n, b1)


def _head_body(h4_ref, a4_ref, deg_ref, ypad_ref, w2s_ref, w2n_ref, b2_ref,
               cb_ref, wd1_ref, bd1_ref, wd2_ref, bd2_ref,
               ze_ref, zq_ref, idx_ref, rec_ref, ssq_ref, sab_ref):
    hp = lambda a, b: jnp.dot(a, b, preferred_element_type=jnp.float32)
    h = jnp.concatenate([h4_ref[j] for j in range(8)], axis=1)
    agg = jnp.concatenate([a4_ref[j] for j in range(8)], axis=1)
    agg = agg / deg_ref[...]
    z_e = hp(h, w2s_ref[...]) + hp(agg, w2n_ref[...]) + b2_ref[...]
    cb = cb_ref[...]
    d = (jnp.sum(z_e * z_e, axis=1, keepdims=True)
         + jnp.sum(cb * cb, axis=1)[None, :]
         - 2.0 * lax.dot_general(z_e, cb, (((1,), (1,)), ((), ())),
                                 preferred_element_type=jnp.float32))
    mind = jnp.min(d, axis=1, keepdims=True)
    iota = lax.broadcasted_iota(jnp.int32, d.shape, 1)
    idx = jnp.min(jnp.where(d == mind, iota, K), axis=1, keepdims=True)
    onehot = (iota == idx).astype(jnp.float32)
    z_q = hp(onehot, cb)
    hd = jnp.maximum(hp(z_q, wd1_ref[...]) + bd1_ref[...], 0.0)
    rec = hp(hd, wd2_ref[...]) + bd2_ref[...]
    ze_ref[...] = z_e
    zq_ref[...] = z_q
    idx_ref[...] = idx
    rec_ref[...] = rec

    @pl.when(pl.program_id(0) == 0)
    def _():
        ssq_ref[...] = jnp.zeros_like(ssq_ref)
        sab_ref[...] = jnp.zeros_like(sab_ref)

    dz = z_q - z_e
    ssq_ref[...] += jnp.sum(dz * dz).reshape(1, 1)
    sab_ref[...] += jnp.sum(jnp.abs(rec - ypad_ref[...])).reshape(1, 1)


def _head(h4, a4, deg, ypad, w2s, w2n, b2, cb, wd1, bd1, wd2p, bd2p):
    full = lambda *shape: pl.BlockSpec(shape, lambda i: tuple(0 for _ in shape))
    return pl.pallas_call(
        _head_body,
        grid=(N // _BN,),
        in_specs=[
            pl.BlockSpec((8, _BN, 16), lambda i: (0, i, 0)),
            pl.BlockSpec((8, _BN, 16), lambda i: (0, i, 0)),
            pl.BlockSpec((_BN, 1), lambda i: (i, 0)),
            pl.BlockSpec((_BN, 16), lambda i: (i, 0)),
            full(D_LAT, D_LAT),
            full(D_LAT, D_LAT),
            full(1, D_LAT),
            full(K, D_LAT),
            full(D_LAT, D_LAT),
            full(1, D_LAT),
            full(D_LAT, 16),
            full(1, 16),
        ],
        out_specs=[
            pl.BlockSpec((_BN, D_LAT), lambda i: (i, 0)),
            pl.BlockSpec((_BN, D_LAT), lambda i: (i, 0)),
            pl.BlockSpec((_BN, 1), lambda i: (i, 0)),
            pl.BlockSpec((_BN, 16), lambda i: (i, 0)),
            pl.BlockSpec((1, 1), lambda i: (0, 0)),
            pl.BlockSpec((1, 1), lambda i: (0, 0)),
        ],
        out_shape=[
            jax.ShapeDtypeStruct((N, D_LAT), jnp.float32),
            jax.ShapeDtypeStruct((N, D_LAT), jnp.float32),
            jax.ShapeDtypeStruct((N, 1), jnp.int32),
            jax.ShapeDtypeStruct((N, 16), jnp.float32),
            jax.ShapeDtypeStruct((1, 1), jnp.float32),
            jax.ShapeDtypeStruct((1, 1), jnp.float32),
        ],
    )(h4, a4, deg, ypad, w2s, w2n, b2, cb, wd1, bd1, wd2p, bd2p)


def kernel(x, edge_index, y, W1_self, W1_neigh, b1, W2_self, W2_neigh, b2,
           codebook, Wd1, bd1, Wd2, bd2):
    f32 = jnp.float32
    src = edge_index[0]
    dst = edge_index[1]
    src_p = jnp.concatenate([src, jnp.zeros((EPAD,), jnp.int32)])
    dst3d = jnp.concatenate(
        [dst, jnp.full((EPAD,), TRASH, jnp.int32)]).reshape(EB, 8, 128)

    xpad = jnp.concatenate(
        [x, jnp.ones((N, 1), f32), jnp.zeros((N, 6), f32)], axis=1)
    ypad = jnp.concatenate([y, jnp.zeros((N, 7), f32)], axis=1)
    w1s = jnp.concatenate([W1_self, jnp.zeros((7, D_LAT), f32)], axis=0)
    w1n = jnp.concatenate([W1_neigh, jnp.zeros((7, D_LAT), f32)], axis=0)
    wd2p = jnp.concatenate([Wd2, jnp.zeros((D_LAT, 7), f32)], axis=1)
    bd2p = jnp.concatenate([bd2, jnp.zeros((7,), f32)])
    zeros16 = jnp.zeros((NA, 16), f32)

    parts = _seg16(xpad, src_p, dst3d, zeros16).reshape(2, NA, 16)
    h4, deg = _enc(xpad, parts, w1s, w1n, b1[None, :])
    a4 = _seg32(h4.reshape(8 * N, 16), src_p, dst3d, zeros16).reshape(8, NA, 16)
    z_e, z_q, idxo, rec16, ssq, sab = _head(
        h4, a4, deg, ypad, W2_self, W2_neigh, b2[None, :], codebook,
        Wd1, bd1[None, :], wd2p, bd2p[None, :])

    recon = rec16[:, :D_IN]
    indices = idxo[:, 0]
    vq_loss = (1.0 + BETA) * ssq[0, 0] / (N * D_LAT)
    recon_loss = sab[0, 0] / (N * D_IN)
    total_loss = recon_loss + vq_loss
    return recon, vq_loss, recon_loss, total_loss, indices, z_e, z_q


# speedup vs baseline: 4.5482x; 1.2249x over previous
import functools

import jax
import jax.numpy as jnp
from jax import lax
from jax.experimental import pallas as pl
from jax.experimental.pallas import tpu as pltpu
from jax.experimental.pallas import tpu_sc as plsc

N = 50000
E = 800000
D_IN = 9
D_LAT = 128
K = 1024
BETA = 0.25

NC = 2
NS = 16
NA = 50048
TRASH = 50040
EROWS = 6400
EPAD = EROWS * 128 - E
EB = EROWS // 8

_mesh = plsc.VectorSubcoreMesh(core_axis_name="c", subcore_axis_name="s",
                               num_cores=NC, num_subcores=NS)


_SEG_SCRATCH = [
    pltpu.VMEM_SHARED((NA, 16), jnp.float32),
    pltpu.VMEM((1024,), jnp.int32),
    pltpu.VMEM((1024,), jnp.int32),
    pltpu.VMEM((8, 128), jnp.int32),
    pltpu.VMEM((8, 128), jnp.int32),
    pltpu.VMEM((8, 128, 16), jnp.float32),
    pltpu.VMEM((8, 128, 16), jnp.float32),
    pltpu.SemaphoreType.DMA,
    pltpu.SemaphoreType.DMA,
    pltpu.SemaphoreType.DMA,
    pltpu.SemaphoreType.DMA,
]


def _edge_pass(table_hbm, src_hbm, dst_hbm, zeros_hbm, acc, sets,
               bb0, npairs, tail, off):
    def load_and_gather(si, bb):
        sbuf, dbuf, rows, gsem, _ = sets[si]
        pltpu.sync_copy(src_hbm.at[pl.ds(bb * 1024, 1024)], sbuf)
        pltpu.sync_copy(dst_hbm.at[bb], dbuf)
        if off is not None:
            offv = jnp.full((16,), off, jnp.int32)
            for i in range(64):
                sl = pl.ds(i * 16, 16)
                sbuf[sl] = sbuf[sl] + offv
        for j in range(8):
            pltpu.async_copy(table_hbm.at[sbuf.at[pl.ds(j * 128, 128)]],
                             rows.at[j], gsem)

    def drain(si, which):
        _, _, rows, gsem, ssem = sets[si]
        sem = gsem if which == "g" else ssem
        for j in range(8):
            pltpu.make_async_copy(zeros_hbm.at[pl.ds(0, 128)],
                                  rows.at[j], sem).wait()

    def fire_scatter(si):
        _, dbuf, rows, _, ssem = sets[si]
        for j in range(8):
            pltpu.async_copy(rows.at[j], acc.at[dbuf.at[j]], ssem, add=True)

    load_and_gather(0, bb0)

    def body(i, carry):
        @pl.when(i > 0)
        def _():
            drain(1, "s")
        load_and_gather(1, bb0 + 2 * i + 1)
        drain(0, "g")
        fire_scatter(0)
        drain(0, "s")
        @pl.when(i < npairs - 1)
        def _():
            load_and_gather(0, bb0 + 2 * i + 2)
        drain(1, "g")
        fire_scatter(1)
        return carry

    lax.fori_loop(0, npairs, body, 0)
    drain(1, "s")
    if tail:
        load_and_gather(0, bb0 + 2 * npairs)
        drain(0, "g")
        fire_scatter(0)
        drain(0, "s")


@functools.partial(
    pl.kernel,
    out_type=jax.ShapeDtypeStruct((2 * NA, 16), jnp.float32),
    mesh=_mesh,
    compiler_params=pltpu.CompilerParams(use_tc_tiling_on_sc=False),
    scratch_types=_SEG_SCRATCH,
)
def _seg16(xpad_hbm, src_hbm, dst_hbm, zeros_hbm, out_hbm,
           acc, sbuf0, sbuf1, dbuf0, dbuf1, rows0, rows1,
           gsem0, gsem1, ssem0, ssem1):
    c = lax.axis_index("c")
    t = lax.axis_index("s")
    sets = ((sbuf0, dbuf0, rows0, gsem0, ssem0),
            (sbuf1, dbuf1, rows1, gsem1, ssem1))
    pltpu.sync_copy(zeros_hbm.at[pl.ds(t * (NA // NS), NA // NS)],
                    acc.at[pl.ds(t * (NA // NS), NA // NS)])
    plsc.subcore_barrier()

    bb0 = c * (EB // 2) + t * (EB // 2 // NS)
    _edge_pass(xpad_hbm, src_hbm, dst_hbm, zeros_hbm, acc, sets,
               bb0, npairs=12, tail=True, off=None)

    plsc.subcore_barrier()
    pltpu.sync_copy(acc.at[pl.ds(t * (NA // NS), NA // NS)],
                    out_hbm.at[pl.ds(c * NA + t * (NA // NS), NA // NS)])


@functools.partial(
    pl.kernel,
    out_type=jax.ShapeDtypeStruct((8 * NA, 16), jnp.float32),
    mesh=_mesh,
    compiler_params=pltpu.CompilerParams(use_tc_tiling_on_sc=False),
    scratch_types=_SEG_SCRATCH,
)
def _seg32(hcat_hbm, src_hbm, dst_hbm, zeros_hbm, out_hbm,
           acc, sbuf0, sbuf1, dbuf0, dbuf1, rows0, rows1,
           gsem0, gsem1, ssem0, ssem1):
    c = lax.axis_index("c")
    t = lax.axis_index("s")
    sets = ((sbuf0, dbuf0, rows0, gsem0, ssem0),
            (sbuf1, dbuf1, rows1, gsem1, ssem1))
    for k in range(4):
        chunk = c * 4 + k
        off = chunk * N
        pltpu.sync_copy(zeros_hbm.at[pl.ds(t * (NA // NS), NA // NS)],
                        acc.at[pl.ds(t * (NA // NS), NA // NS)])
        plsc.subcore_barrier()

        _edge_pass(hcat_hbm, src_hbm, dst_hbm, zeros_hbm, acc, sets,
                   t * (EB // NS), npairs=25, tail=False, off=off)

        plsc.subcore_barrier()
        pltpu.sync_copy(acc.at[pl.ds(t * (NA // NS), NA // NS)],
                        out_hbm.at[pl.ds(chunk * NA + t * (NA // NS), NA // NS)])
        plsc.subcore_barrier()


_BN = 400


def _enc_body(xp_ref, p_ref, w1s_ref, w1n_ref, b1_ref, h4_ref, deg_ref):
    p = p_ref[0] + p_ref[1]
    deg = jnp.maximum(p[:, 9:10], 1.0)
    agg = p / deg
    hp = (jnp.dot(xp_ref[...], w1s_ref[...], preferred_element_type=jnp.float32)
          + jnp.dot(agg, w1n_ref[...], preferred_element_type=jnp.float32)
          + b1_ref[...])
    h = jnp.maximum(hp, 0.0)
    h4_ref[...] = jnp.stack([h[:, 16 * j:16 * j + 16] for j in range(8)])
    deg_ref[...] = deg


def _enc(xpad, parts, w1s, w1n, b1):
    return pl.pallas_call(
        _enc_body,
        grid=(N // _BN,),
        in_specs=[
            pl.BlockSpec((_BN, 16), lambda i: (i, 0)),
            pl.BlockSpec((2, _BN, 16), lambda i: (0, i, 0)),
            pl.BlockSpec((16, D_LAT), lambda i: (0, 0)),
            pl.BlockSpec((16, D_LAT), lambda i: (0, 0)),
            pl.BlockSpec((1, D_LAT), lambda i: (0, 0)),
        ],
        out_specs=[
            pl.BlockSpec((8, _BN, 16), lambda i: (0, i, 0)),
            pl.BlockSpec((_BN, 1), lambda i: (i, 0)),
        ],
        out_shape=[
            jax.ShapeDtypeStruct((8, N, 16), jnp.float32),
            jax.ShapeDtypeStruct((N, 1), jnp.float32),
        ],
    )(xpad, parts, w1s, w1n, b1)


def _head_body(h4_ref, a4_ref, deg_ref, ypad_ref, w2s_ref, w2n_ref, b2_ref,
               cb_ref, wd1_ref, bd1_ref, wd2_ref, bd2_ref,
               ze_ref, zq_ref, idx_ref, rec_ref, ssq_ref, sab_ref):
    hp = lambda a, b: jnp.dot(a, b, preferred_element_type=jnp.float32)
    h = jnp.concatenate([h4_ref[j] for j in range(8)], axis=1)
    agg = jnp.concatenate([a4_ref[j] for j in range(8)], axis=1)
    agg = agg / deg_ref[...]
    z_e = hp(h, w2s_ref[...]) + hp(agg, w2n_ref[...]) + b2_ref[...]
    cb = cb_ref[...]
    d = (jnp.sum(z_e * z_e, axis=1, keepdims=True)
         + jnp.sum(cb * cb, axis=1)[None, :]
         - 2.0 * lax.dot_general(z_e, cb, (((1,), (1,)), ((), ())),
                                 preferred_element_type=jnp.float32))
    mind = jnp.min(d, axis=1, keepdims=True)
    iota = lax.broadcasted_iota(jnp.int32, d.shape, 1)
    idx = jnp.min(jnp.where(d == mind, iota, K), axis=1, keepdims=True)
    onehot = (iota == idx).astype(jnp.float32)
    z_q = hp(onehot, cb)
    hd = jnp.maximum(hp(z_q, wd1_ref[...]) + bd1_ref[...], 0.0)
    rec = hp(hd, wd2_ref[...]) + bd2_ref[...]
    ze_ref[...] = z_e
    zq_ref[...] = z_q
    idx_ref[...] = idx
    rec_ref[...] = rec

    @pl.when(pl.program_id(0) == 0)
    def _():
        ssq_ref[...] = jnp.zeros_like(ssq_ref)
        sab_ref[...] = jnp.zeros_like(sab_ref)

    dz = z_q - z_e
    ssq_ref[...] += jnp.sum(dz * dz).reshape(1, 1)
    sab_ref[...] += jnp.sum(jnp.abs(rec - ypad_ref[...])).reshape(1, 1)


def _head(h4, a4, deg, ypad, w2s, w2n, b2, cb, wd1, bd1, wd2p, bd2p):
    full = lambda *shape: pl.BlockSpec(shape, lambda i: tuple(0 for _ in shape))
    return pl.pallas_call(
        _head_body,
        grid=(N // _BN,),
        in_specs=[
            pl.BlockSpec((8, _BN, 16), lambda i: (0, i, 0)),
            pl.BlockSpec((8, _BN, 16), lambda i: (0, i, 0)),
            pl.BlockSpec((_BN, 1), lambda i: (i, 0)),
            pl.BlockSpec((_BN, 16), lambda i: (i, 0)),
            full(D_LAT, D_LAT),
            full(D_LAT, D_LAT),
            full(1, D_LAT),
            full(K, D_LAT),
            full(D_LAT, D_LAT),
            full(1, D_LAT),
            full(D_LAT, 16),
            full(1, 16),
        ],
        out_specs=[
            pl.BlockSpec((_BN, D_LAT), lambda i: (i, 0)),
            pl.BlockSpec((_BN, D_LAT), lambda i: (i, 0)),
            pl.BlockSpec((_BN, 1), lambda i: (i, 0)),
            pl.BlockSpec((_BN, 16), lambda i: (i, 0)),
            pl.BlockSpec((1, 1), lambda i: (0, 0)),
            pl.BlockSpec((1, 1), lambda i: (0, 0)),
        ],
        out_shape=[
            jax.ShapeDtypeStruct((N, D_LAT), jnp.float32),
            jax.ShapeDtypeStruct((N, D_LAT), jnp.float32),
            jax.ShapeDtypeStruct((N, 1), jnp.int32),
            jax.ShapeDtypeStruct((N, 16), jnp.float32),
            jax.ShapeDtypeStruct((1, 1), jnp.float32),
            jax.ShapeDtypeStruct((1, 1), jnp.float32),
        ],
    )(h4, a4, deg, ypad, w2s, w2n, b2, cb, wd1, bd1, wd2p, bd2p)


def kernel(x, edge_index, y, W1_self, W1_neigh, b1, W2_self, W2_neigh, b2,
           codebook, Wd1, bd1, Wd2, bd2):
    f32 = jnp.float32
    src = edge_index[0]
    dst = edge_index[1]
    src_p = jnp.concatenate([src, jnp.zeros((EPAD,), jnp.int32)])
    dst3d = jnp.concatenate(
        [dst, jnp.full((EPAD,), TRASH, jnp.int32)]).reshape(EB, 8, 128)

    xpad = jnp.concatenate(
        [x, jnp.ones((N, 1), f32), jnp.zeros((N, 6), f32)], axis=1)
    ypad = jnp.concatenate([y, jnp.zeros((N, 7), f32)], axis=1)
    w1s = jnp.concatenate([W1_self, jnp.zeros((7, D_LAT), f32)], axis=0)
    w1n = jnp.concatenate([W1_neigh, jnp.zeros((7, D_LAT), f32)], axis=0)
    wd2p = jnp.concatenate([Wd2, jnp.zeros((D_LAT, 7), f32)], axis=1)
    bd2p = jnp.concatenate([bd2, jnp.zeros((7,), f32)])
    zeros16 = jnp.zeros((NA, 16), f32)

    parts = _seg16(xpad, src_p, dst3d, zeros16).reshape(2, NA, 16)
    h4, deg = _enc(xpad, parts, w1s, w1n, b1[None, :])
    a4 = _seg32(h4.reshape(8 * N, 16), src_p, dst3d, zeros16).reshape(8, NA, 16)
    z_e, z_q, idxo, rec16, ssq, sab = _head(
        h4, a4, deg, ypad, W2_self, W2_neigh, b2[None, :], codebook,
        Wd1, bd1[None, :], wd2p, bd2p[None, :])

    recon = rec16[:, :D_IN]
    indices = idxo[:, 0]
    vq_loss = (1.0 + BETA) * ssq[0, 0] / (N * D_LAT)
    recon_loss = sab[0, 0] / (N * D_IN)
    total_loss = recon_loss + vq_loss
    return recon, vq_loss, recon_loss, total_loss, indices, z_e, z_q


# speedup vs baseline: 4.6039x; 1.0122x over previous
import functools

import jax
import jax.numpy as jnp
from jax import lax
from jax.experimental import pallas as pl
from jax.experimental.pallas import tpu as pltpu
from jax.experimental.pallas import tpu_sc as plsc

N = 50000
E = 800000
D_IN = 9
D_LAT = 128
K = 1024
BETA = 0.25

NC = 2
NS = 16
NA = 53248
EROWS = 6400
EPAD = EROWS * 128 - E

_mesh = plsc.VectorSubcoreMesh(core_axis_name="c", subcore_axis_name="s",
                               num_cores=NC, num_subcores=NS)


def _seg_scratch(R):
    return [
        pltpu.VMEM_SHARED((NA, 16), jnp.float32),
        pltpu.VMEM((R * 128,), jnp.int32),
        pltpu.VMEM((R * 128,), jnp.int32),
        pltpu.VMEM((R, 128), jnp.int32),
        pltpu.VMEM((R, 128), jnp.int32),
        pltpu.VMEM((R, 128, 16), jnp.float32),
        pltpu.VMEM((R, 128, 16), jnp.float32),
        pltpu.SemaphoreType.DMA,
        pltpu.SemaphoreType.DMA,
        pltpu.SemaphoreType.DMA,
        pltpu.SemaphoreType.DMA,
    ]


def _edge_pass(table_hbm, src_hbm, dst_hbm, zeros_hbm, acc, sets,
               R, row0, npairs, tail, off):
    def load_and_gather(si, b):
        sbuf, dbuf, rows, gsem, _ = sets[si]
        r = row0 + b * R
        pltpu.sync_copy(src_hbm.at[pl.ds(r * 128, R * 128)], sbuf)
        pltpu.sync_copy(dst_hbm.at[pl.ds(r, R)], dbuf)
        if off is not None:
            offv = jnp.full((16,), off, jnp.int32)
            for i in range(R * 8):
                sl = pl.ds(i * 16, 16)
                sbuf[sl] = sbuf[sl] + offv
        for j in range(R):
            pltpu.async_copy(table_hbm.at[sbuf.at[pl.ds(j * 128, 128)]],
                             rows.at[j], gsem)

    def drain(si, which):
        _, _, rows, gsem, ssem = sets[si]
        sem = gsem if which == "g" else ssem
        for j in range(R):
            pltpu.make_async_copy(zeros_hbm.at[pl.ds(0, 128)],
                                  rows.at[j], sem).wait()

    def fire_scatter(si):
        _, dbuf, rows, _, ssem = sets[si]
        for j in range(R):
            pltpu.async_copy(rows.at[j], acc.at[dbuf.at[j]], ssem, add=True)

    load_and_gather(0, 0)

    def body(i, carry):
        @pl.when(i > 0)
        def _():
            drain(1, "s")
        load_and_gather(1, 2 * i + 1)
        drain(0, "g")
        fire_scatter(0)
        drain(0, "s")
        @pl.when(i < npairs - 1)
        def _():
            load_and_gather(0, 2 * i + 2)
        drain(1, "g")
        fire_scatter(1)
        return carry

    lax.fori_loop(0, npairs, body, 0)
    drain(1, "s")
    if tail:
        load_and_gather(0, 2 * npairs)
        drain(0, "g")
        fire_scatter(0)
        drain(0, "s")


@functools.partial(
    pl.kernel,
    out_type=jax.ShapeDtypeStruct((2 * NA, 16), jnp.float32),
    mesh=_mesh,
    compiler_params=pltpu.CompilerParams(use_tc_tiling_on_sc=False),
    scratch_types=_seg_scratch(8),
)
def _seg16(xpad_hbm, src_hbm, dst_hbm, zeros_hbm, out_hbm,
           acc, sbuf0, sbuf1, dbuf0, dbuf1, rows0, rows1,
           gsem0, gsem1, ssem0, ssem1):
    c = lax.axis_index("c")
    t = lax.axis_index("s")
    sets = ((sbuf0, dbuf0, rows0, gsem0, ssem0),
            (sbuf1, dbuf1, rows1, gsem1, ssem1))
    pltpu.sync_copy(zeros_hbm.at[pl.ds(t * (NA // NS), NA // NS)],
                    acc.at[pl.ds(t * (NA // NS), NA // NS)])
    plsc.subcore_barrier()

    row0 = c * (EROWS // 2) + t * (EROWS // 2 // NS)
    _edge_pass(xpad_hbm, src_hbm, dst_hbm, zeros_hbm, acc, sets,
               R=8, row0=row0, npairs=12, tail=True, off=None)

    plsc.subcore_barrier()
    pltpu.sync_copy(acc.at[pl.ds(t * (NA // NS), NA // NS)],
                    out_hbm.at[pl.ds(c * NA + t * (NA // NS), NA // NS)])


@functools.partial(
    pl.kernel,
    out_type=jax.ShapeDtypeStruct((8 * NA, 16), jnp.float32),
    mesh=_mesh,
    compiler_params=pltpu.CompilerParams(use_tc_tiling_on_sc=False),
    scratch_types=_seg_scratch(16),
)
def _seg32(hcat_hbm, src_hbm, dst_hbm, zeros_hbm, out_hbm,
           acc, sbuf0, sbuf1, dbuf0, dbuf1, rows0, rows1,
           gsem0, gsem1, ssem0, ssem1):
    c = lax.axis_index("c")
    t = lax.axis_index("s")
    sets = ((sbuf0, dbuf0, rows0, gsem0, ssem0),
            (sbuf1, dbuf1, rows1, gsem1, ssem1))
    for k in range(4):
        chunk = c * 4 + k
        off = chunk * N
        pltpu.sync_copy(zeros_hbm.at[pl.ds(t * (NA // NS), NA // NS)],
                        acc.at[pl.ds(t * (NA // NS), NA // NS)])
        plsc.subcore_barrier()

        _edge_pass(hcat_hbm, src_hbm, dst_hbm, zeros_hbm, acc, sets,
                   R=16, row0=t * (EROWS // NS), npairs=12, tail=True, off=off)

        plsc.subcore_barrier()
        pltpu.sync_copy(acc.at[pl.ds(t * (NA // NS), NA // NS)],
                        out_hbm.at[pl.ds(chunk * NA + t * (NA // NS), NA // NS)])
        plsc.subcore_barrier()


_BN = 400


def _enc_body(xp_ref, p_ref, w1s_ref, w1n_ref, b1_ref, h4_ref, deg_ref):
    p = p_ref[0] + p_ref[1]
    deg = jnp.maximum(p[:, 9:10], 1.0)
    agg = p / deg
    hp = (jnp.dot(xp_ref[...], w1s_ref[...], preferred_element_type=jnp.float32)
          + jnp.dot(agg, w1n_ref[...], preferred_element_type=jnp.float32)
          + b1_ref[...])
    h = jnp.maximum(hp, 0.0)
    h4_ref[...] = jnp.stack([h[:, 16 * j:16 * j + 16] for j in range(8)])
    deg_ref[...] = deg


def _enc(xpad, parts, w1s, w1n, b1):
    return pl.pallas_call(
        _enc_body,
        grid=(N // _BN,),
        in_specs=[
            pl.BlockSpec((_BN, 16), lambda i: (i, 0)),
            pl.BlockSpec((2, _BN, 16), lambda i: (0, i, 0)),
            pl.BlockSpec((16, D_LAT), lambda i: (0, 0)),
            pl.BlockSpec((16, D_LAT), lambda i: (0, 0)),
            pl.BlockSpec((1, D_LAT), lambda i: (0, 0)),
        ],
        out_specs=[
            pl.BlockSpec((8, _BN, 16), lambda i: (0, i, 0)),
            pl.BlockSpec((_BN, 1), lambda i: (i, 0)),
        ],
        out_shape=[
            jax.ShapeDtypeStruct((8, N, 16), jnp.float32),
            jax.ShapeDtypeStruct((N, 1), jnp.float32),
        ],
    )(xpad, parts, w1s, w1n, b1)


def _head_body(h4_ref, a4_ref, deg_ref, ypad_ref, w2s_ref, w2n_ref, b2_ref,
               cb_ref, wd1_ref, bd1_ref, wd2_ref, bd2_ref,
               ze_ref, zq_ref, idx_ref, rec_ref, ssq_ref, sab_ref):
    hp = lambda a, b: jnp.dot(a, b, preferred_element_type=jnp.float32)
    h = jnp.concatenate([h4_ref[j] for j in range(8)], axis=1)
    agg = jnp.concatenate([a4_ref[j] for j in range(8)], axis=1)
    agg = agg / deg_ref[...]
    z_e = hp(h, w2s_ref[...]) + hp(agg, w2n_ref[...]) + b2_ref[...]
    cb = cb_ref[...]
    d = (jnp.sum(z_e * z_e, axis=1, keepdims=True)
         + jnp.sum(cb * cb, axis=1)[None, :]
         - 2.0 * lax.dot_general(z_e, cb, (((1,), (1,)), ((), ())),
                                 preferred_element_type=jnp.float32))
    mind = jnp.min(d, axis=1, keepdims=True)
    iota = lax.broadcasted_iota(jnp.int32, d.shape, 1)
    idx = jnp.min(jnp.where(d == mind, iota, K), axis=1, keepdims=True)
    onehot = (iota == idx).astype(jnp.float32)
    z_q = hp(onehot, cb)
    hd = jnp.maximum(hp(z_q, wd1_ref[...]) + bd1_ref[...], 0.0)
    rec = hp(hd, wd2_ref[...]) + bd2_ref[...]
    ze_ref[...] = z_e
    zq_ref[...] = z_q
    idx_ref[...] = idx
    rec_ref[...] = rec

    @pl.when(pl.program_id(0) == 0)
    def _():
        ssq_ref[...] = jnp.zeros_like(ssq_ref)
        sab_ref[...] = jnp.zeros_like(sab_ref)

    dz = z_q - z_e
    ssq_ref[...] += jnp.sum(dz * dz).reshape(1, 1)
    sab_ref[...] += jnp.sum(jnp.abs(rec - ypad_ref[...])).reshape(1, 1)


def _head(h4, a4, deg, ypad, w2s, w2n, b2, cb, wd1, bd1, wd2p, bd2p):
    full = lambda *shape: pl.BlockSpec(shape, lambda i: tuple(0 for _ in shape))
    return pl.pallas_call(
        _head_body,
        grid=(N // _BN,),
        in_specs=[
            pl.BlockSpec((8, _BN, 16), lambda i: (0, i, 0)),
            pl.BlockSpec((8, _BN, 16), lambda i: (0, i, 0)),
            pl.BlockSpec((_BN, 1), lambda i: (i, 0)),
            pl.BlockSpec((_BN, 16), lambda i: (i, 0)),
            full(D_LAT, D_LAT),
            full(D_LAT, D_LAT),
            full(1, D_LAT),
            full(K, D_LAT),
            full(D_LAT, D_LAT),
            full(1, D_LAT),
            full(D_LAT, 16),
            full(1, 16),
        ],
        out_specs=[
            pl.BlockSpec((_BN, D_LAT), lambda i: (i, 0)),
            pl.BlockSpec((_BN, D_LAT), lambda i: (i, 0)),
            pl.BlockSpec((_BN, 1), lambda i: (i, 0)),
            pl.BlockSpec((_BN, 16), lambda i: (i, 0)),
            pl.BlockSpec((1, 1), lambda i: (0, 0)),
            pl.BlockSpec((1, 1), lambda i: (0, 0)),
        ],
        out_shape=[
            jax.ShapeDtypeStruct((N, D_LAT), jnp.float32),
            jax.ShapeDtypeStruct((N, D_LAT), jnp.float32),
            jax.ShapeDtypeStruct((N, 1), jnp.int32),
            jax.ShapeDtypeStruct((N, 16), jnp.float32),
            jax.ShapeDtypeStruct((1, 1), jnp.float32),
            jax.ShapeDtypeStruct((1, 1), jnp.float32),
        ],
    )(h4, a4, deg, ypad, w2s, w2n, b2, cb, wd1, bd1, wd2p, bd2p)


def kernel(x, edge_index, y, W1_self, W1_neigh, b1, W2_self, W2_neigh, b2,
           codebook, Wd1, bd1, Wd2, bd2):
    f32 = jnp.float32
    src = edge_index[0]
    dst = edge_index[1]
    src_p = jnp.concatenate([src, jnp.zeros((EPAD,), jnp.int32)])
    trash = N + 48 + (jnp.arange(EPAD, dtype=jnp.int32) % 3200)
    dst2d = jnp.concatenate([dst, trash]).reshape(EROWS, 128)

    xpad = jnp.concatenate(
        [x, jnp.ones((N, 1), f32), jnp.zeros((N, 6), f32)], axis=1)
    ypad = jnp.concatenate([y, jnp.zeros((N, 7), f32)], axis=1)
    w1s = jnp.concatenate([W1_self, jnp.zeros((7, D_LAT), f32)], axis=0)
    w1n = jnp.concatenate([W1_neigh, jnp.zeros((7, D_LAT), f32)], axis=0)
    wd2p = jnp.concatenate([Wd2, jnp.zeros((D_LAT, 7), f32)], axis=1)
    bd2p = jnp.concatenate([bd2, jnp.zeros((7,), f32)])
    zeros16 = jnp.zeros((NA, 16), f32)

    parts = _seg16(xpad, src_p, dst2d, zeros16).reshape(2, NA, 16)
    h4, deg = _enc(xpad, parts, w1s, w1n, b1[None, :])
    a4 = _seg32(h4.reshape(8 * N, 16), src_p, dst2d, zeros16).reshape(8, NA, 16)
    z_e, z_q, idxo, rec16, ssq, sab = _head(
        h4, a4, deg, ypad, W2_self, W2_neigh, b2[None, :], codebook,
        Wd1, bd1[None, :], wd2p, bd2p[None, :])

    recon = rec16[:, :D_IN]
    indices = idxo[:, 0]
    vq_loss = (1.0 + BETA) * ssq[0, 0] / (N * D_LAT)
    recon_loss = sab[0, 0] / (N * D_IN)
    total_loss = recon_loss + vq_loss
    return recon, vq_loss, recon_loss, total_loss, indices, z_e, z_q


# speedup vs baseline: 5.1164x; 1.1113x over previous
import functools

import jax
import jax.numpy as jnp
from jax import lax
from jax.experimental import pallas as pl
from jax.experimental.pallas import tpu as pltpu
from jax.experimental.pallas import tpu_sc as plsc

N = 50000
E = 800000
D_IN = 9
D_LAT = 128
K = 1024
BETA = 0.25

NC = 2
NS = 16
NA = 50048
EROWS = 6400
EPAD = EROWS * 128 - E

_mesh = plsc.VectorSubcoreMesh(core_axis_name="c", subcore_axis_name="s",
                               num_cores=NC, num_subcores=NS)


def _seg_scratch(R):
    return [
        pltpu.VMEM_SHARED((NA, 16), jnp.float32),
        pltpu.VMEM((R * 128,), jnp.int32),
        pltpu.VMEM((R * 128,), jnp.int32),
        pltpu.VMEM((R, 128), jnp.int32),
        pltpu.VMEM((R, 128), jnp.int32),
        pltpu.VMEM((R, 128, 16), jnp.float32),
        pltpu.VMEM((R, 128, 16), jnp.float32),
        pltpu.SemaphoreType.DMA,
        pltpu.SemaphoreType.DMA,
        pltpu.SemaphoreType.DMA,
        pltpu.SemaphoreType.DMA,
    ]


def _edge_pass(table_hbm, src_hbm, dst_hbm, zeros_hbm, acc, sets,
               R, row0, npairs, tail, off):
    def load_and_gather(si, b):
        sbuf, dbuf, rows, gsem, _ = sets[si]
        r = row0 + b * R
        pltpu.sync_copy(src_hbm.at[pl.ds(r * 128, R * 128)], sbuf)
        pltpu.sync_copy(dst_hbm.at[pl.ds(r, R)], dbuf)
        if off is not None:
            offv = jnp.full((16,), off, jnp.int32)
            for i in range(R * 8):
                sl = pl.ds(i * 16, 16)
                sbuf[sl] = sbuf[sl] + offv
        for j in range(R):
            pltpu.async_copy(table_hbm.at[sbuf.at[pl.ds(j * 128, 128)]],
                             rows.at[j], gsem)

    def drain(si, which):
        _, _, rows, gsem, ssem = sets[si]
        sem = gsem if which == "g" else ssem
        for j in range(R):
            pltpu.make_async_copy(zeros_hbm.at[pl.ds(0, 128)],
                                  rows.at[j], sem).wait()

    def fire_scatter(si):
        _, dbuf, rows, _, ssem = sets[si]
        for j in range(R):
            pltpu.async_copy(rows.at[j], acc.at[dbuf.at[j]], ssem, add=True)

    load_and_gather(0, 0)

    def body(i, carry):
        @pl.when(i > 0)
        def _():
            drain(1, "s")
        load_and_gather(1, 2 * i + 1)
        drain(0, "g")
        fire_scatter(0)
        drain(0, "s")
        @pl.when(i < npairs - 1)
        def _():
            load_and_gather(0, 2 * i + 2)
        drain(1, "g")
        fire_scatter(1)
        return carry

    lax.fori_loop(0, npairs, body, 0)
    drain(1, "s")
    if tail:
        load_and_gather(0, 2 * npairs)
        drain(0, "g")
        fire_scatter(0)
        drain(0, "s")


@functools.partial(
    pl.kernel,
    out_type=jax.ShapeDtypeStruct((2 * NA, 16), jnp.float32),
    mesh=_mesh,
    compiler_params=pltpu.CompilerParams(use_tc_tiling_on_sc=False),
    scratch_types=_seg_scratch(8),
)
def _seg16(xpad_hbm, src_hbm, dst_hbm, zeros_hbm, out_hbm,
           acc, sbuf0, sbuf1, dbuf0, dbuf1, rows0, rows1,
           gsem0, gsem1, ssem0, ssem1):
    c = lax.axis_index("c")
    t = lax.axis_index("s")
    sets = ((sbuf0, dbuf0, rows0, gsem0, ssem0),
            (sbuf1, dbuf1, rows1, gsem1, ssem1))
    pltpu.sync_copy(zeros_hbm.at[pl.ds(t * (NA // NS), NA // NS)],
                    acc.at[pl.ds(t * (NA // NS), NA // NS)])
    plsc.subcore_barrier()

    row0 = c * (EROWS // 2) + t * (EROWS // 2 // NS)
    _edge_pass(xpad_hbm, src_hbm, dst_hbm, zeros_hbm, acc, sets,
               R=8, row0=row0, npairs=12, tail=True, off=None)

    plsc.subcore_barrier()
    pltpu.sync_copy(acc.at[pl.ds(t * (NA // NS), NA // NS)],
                    out_hbm.at[pl.ds(c * NA + t * (NA // NS), NA // NS)])


@functools.partial(
    pl.kernel,
    out_type=jax.ShapeDtypeStruct((4 * NA, 32), jnp.float32),
    mesh=_mesh,
    compiler_params=pltpu.CompilerParams(use_tc_tiling_on_sc=False),
    scratch_types=[
        pltpu.VMEM_SHARED((NA, 32), jnp.float32),
        pltpu.VMEM((2048,), jnp.int32),
        pltpu.VMEM((16, 128), jnp.int32),
        pltpu.VMEM((2, 128, 32), jnp.float32),
        pltpu.VMEM((2, 128, 32), jnp.float32),
        pltpu.SemaphoreType.DMA,
        pltpu.SemaphoreType.DMA,
        pltpu.SemaphoreType.DMA,
        pltpu.SemaphoreType.DMA,
    ],
)
def _seg32(hcat_hbm, src_hbm, dst_hbm, zeros_hbm, out_hbm,
           acc, sbuf, dbuf, rowsA, rowsB, gsemA, gsemB, ssemA, ssemB):
    c = lax.axis_index("c")
    t = lax.axis_index("s")
    rows = (rowsA, rowsB)
    gsem = (gsemA, gsemB)
    ssem = (ssemA, ssemB)
    for k in range(2):
        chunk = c * 2 + k
        off = chunk * N
        pltpu.sync_copy(zeros_hbm.at[pl.ds(t * (NA // NS), NA // NS)],
                        acc.at[pl.ds(t * (NA // NS), NA // NS)])
        plsc.subcore_barrier()

        def fire_g(sub):
            bi = sub % 2
            for j in range(2):
                r = 2 * sub + j
                pltpu.async_copy(
                    hcat_hbm.at[sbuf.at[pl.ds(r * 128, 128)]],
                    rows[bi].at[j], gsem[bi])

        def fire_s(sub):
            bi = sub % 2
            for j in range(2):
                r = 2 * sub + j
                pltpu.async_copy(rows[bi].at[j], acc.at[dbuf.at[r]],
                                 ssem[bi], add=True)

        def drain(sub, sems):
            bi = sub % 2
            for j in range(2):
                pltpu.make_async_copy(zeros_hbm.at[pl.ds(0, 128)],
                                      rows[bi].at[j], sems[bi]).wait()

        def body(b, carry):
            r0 = t * (EROWS // NS) + b * 16
            pltpu.sync_copy(src_hbm.at[pl.ds(r0 * 128, 2048)], sbuf)
            pltpu.sync_copy(dst_hbm.at[pl.ds(r0, 16)], dbuf)
            offv = jnp.full((16,), off, jnp.int32)
            for i in range(128):
                sl = pl.ds(i * 16, 16)
                sbuf[sl] = sbuf[sl] + offv
            fire_g(0)
            for sub in range(1, 8):
                fire_g(sub)
                drain(sub - 1, gsem)
                fire_s(sub - 1)
                drain(sub - 1, ssem)
            drain(7, gsem)
            fire_s(7)
            drain(7, ssem)
            return carry

        lax.fori_loop(0, EROWS // NS // 16, body, 0)
        plsc.subcore_barrier()
        pltpu.sync_copy(acc.at[pl.ds(t * (NA // NS), NA // NS)],
                        out_hbm.at[pl.ds(chunk * NA + t * (NA // NS), NA // NS)])
        plsc.subcore_barrier()


_BN = 1000


def _enc_body(xp_ref, p_ref, w1s_ref, w1n_ref, b1_ref, h4_ref, deg_ref):
    p = p_ref[0] + p_ref[1]
    deg = jnp.maximum(p[:, 9:10], 1.0)
    agg = p / deg
    hp = (jnp.dot(xp_ref[...], w1s_ref[...], preferred_element_type=jnp.float32)
          + jnp.dot(agg, w1n_ref[...], preferred_element_type=jnp.float32)
          + b1_ref[...])
    h = jnp.maximum(hp, 0.0)
    h4_ref[...] = jnp.stack([h[:, 32 * j:32 * j + 32] for j in range(4)])
    deg_ref[...] = deg


def _enc(xpad, parts, w1s, w1n, b1):
    return pl.pallas_call(
        _enc_body,
        grid=(N // _BN,),
        in_specs=[
            pl.BlockSpec((_BN, 16), lambda i: (i, 0)),
            pl.BlockSpec((2, _BN, 16), lambda i: (0, i, 0)),
            pl.BlockSpec((16, D_LAT), lambda i: (0, 0)),
            pl.BlockSpec((16, D_LAT), lambda i: (0, 0)),
            pl.BlockSpec((1, D_LAT), lambda i: (0, 0)),
        ],
        out_specs=[
            pl.BlockSpec((4, _BN, 32), lambda i: (0, i, 0)),
            pl.BlockSpec((_BN, 1), lambda i: (i, 0)),
        ],
        out_shape=[
            jax.ShapeDtypeStruct((4, N, 32), jnp.float32),
            jax.ShapeDtypeStruct((N, 1), jnp.float32),
        ],
    )(xpad, parts, w1s, w1n, b1)


def _head_body(h4_ref, a4_ref, deg_ref, ypad_ref, w2s_ref, w2n_ref, b2_ref,
               cb_ref, wd1_ref, bd1_ref, wd2_ref, bd2_ref,
               ze_ref, zq_ref, idx_ref, rec_ref, ssq_ref, sab_ref):
    hp = lambda a, b: jnp.dot(a, b, preferred_element_type=jnp.float32)
    h = jnp.concatenate([h4_ref[j] for j in range(4)], axis=1)
    agg = jnp.concatenate([a4_ref[j] for j in range(4)], axis=1)
    agg = agg / deg_ref[...]
    z_e = hp(h, w2s_ref[...]) + hp(agg, w2n_ref[...]) + b2_ref[...]
    cb = cb_ref[...]
    d = (jnp.sum(z_e * z_e, axis=1, keepdims=True)
         + jnp.sum(cb * cb, axis=1)[None, :]
         - 2.0 * lax.dot_general(z_e, cb, (((1,), (1,)), ((), ())),
                                 preferred_element_type=jnp.float32))
    mind = jnp.min(d, axis=1, keepdims=True)
    iota = lax.broadcasted_iota(jnp.int32, d.shape, 1)
    idx = jnp.min(jnp.where(d == mind, iota, K), axis=1, keepdims=True)
    onehot = (iota == idx).astype(jnp.float32)
    z_q = hp(onehot, cb)
    hd = jnp.maximum(hp(z_q, wd1_ref[...]) + bd1_ref[...], 0.0)
    rec = hp(hd, wd2_ref[...]) + bd2_ref[...]
    ze_ref[...] = z_e
    zq_ref[...] = z_q
    idx_ref[...] = idx
    rec_ref[...] = rec

    @pl.when(pl.program_id(0) == 0)
    def _():
        ssq_ref[...] = jnp.zeros_like(ssq_ref)
        sab_ref[...] = jnp.zeros_like(sab_ref)

    dz = z_q - z_e
    ssq_ref[...] += jnp.sum(dz * dz).reshape(1, 1)
    sab_ref[...] += jnp.sum(jnp.abs(rec - ypad_ref[...])).reshape(1, 1)


def _head(h4, a4, deg, ypad, w2s, w2n, b2, cb, wd1, bd1, wd2p, bd2p):
    full = lambda *shape: pl.BlockSpec(shape, lambda i: tuple(0 for _ in shape))
    return pl.pallas_call(
        _head_body,
        grid=(N // _BN,),
        in_specs=[
            pl.BlockSpec((4, _BN, 32), lambda i: (0, i, 0)),
            pl.BlockSpec((4, _BN, 32), lambda i: (0, i, 0)),
            pl.BlockSpec((_BN, 1), lambda i: (i, 0)),
            pl.BlockSpec((_BN, 16), lambda i: (i, 0)),
            full(D_LAT, D_LAT),
            full(D_LAT, D_LAT),
            full(1, D_LAT),
            full(K, D_LAT),
            full(D_LAT, D_LAT),
            full(1, D_LAT),
            full(D_LAT, 16),
            full(1, 16),
        ],
        out_specs=[
            pl.BlockSpec((_BN, D_LAT), lambda i: (i, 0)),
            pl.BlockSpec((_BN, D_LAT), lambda i: (i, 0)),
            pl.BlockSpec((_BN, 1), lambda i: (i, 0)),
            pl.BlockSpec((_BN, 16), lambda i: (i, 0)),
            pl.BlockSpec((1, 1), lambda i: (0, 0)),
            pl.BlockSpec((1, 1), lambda i: (0, 0)),
        ],
        out_shape=[
            jax.ShapeDtypeStruct((N, D_LAT), jnp.float32),
            jax.ShapeDtypeStruct((N, D_LAT), jnp.float32),
            jax.ShapeDtypeStruct((N, 1), jnp.int32),
            jax.ShapeDtypeStruct((N, 16), jnp.float32),
            jax.ShapeDtypeStruct((1, 1), jnp.float32),
            jax.ShapeDtypeStruct((1, 1), jnp.float32),
        ],
    )(h4, a4, deg, ypad, w2s, w2n, b2, cb, wd1, bd1, wd2p, bd2p)


def kernel(x, edge_index, y, W1_self, W1_neigh, b1, W2_self, W2_neigh, b2,
           codebook, Wd1, bd1, Wd2, bd2):
    f32 = jnp.float32
    src = edge_index[0]
    dst = edge_index[1]
    src_p = jnp.concatenate([src, jnp.zeros((EPAD,), jnp.int32)])
    trash = N + (jnp.arange(EPAD, dtype=jnp.int32) % 48)
    dst2d = jnp.concatenate([dst, trash]).reshape(EROWS, 128)

    xpad = jnp.concatenate(
        [x, jnp.ones((N, 1), f32), jnp.zeros((N, 6), f32)], axis=1)
    ypad = jnp.concatenate([y, jnp.zeros((N, 7), f32)], axis=1)
    w1s = jnp.concatenate([W1_self, jnp.zeros((7, D_LAT), f32)], axis=0)
    w1n = jnp.concatenate([W1_neigh, jnp.zeros((7, D_LAT), f32)], axis=0)
    wd2p = jnp.concatenate([Wd2, jnp.zeros((D_LAT, 7), f32)], axis=1)
    bd2p = jnp.concatenate([bd2, jnp.zeros((7,), f32)])
    zeros16 = jnp.zeros((NA, 16), f32)
    zeros32 = jnp.zeros((NA, 32), f32)

    parts = _seg16(xpad, src_p, dst2d, zeros16).reshape(2, NA, 16)
    h4, deg = _enc(xpad, parts, w1s, w1n, b1[None, :])
    a4 = _seg32(h4.reshape(4 * N, 32), src_p, dst2d, zeros32).reshape(4, NA, 32)
    z_e, z_q, idxo, rec16, ssq, sab = _head(
        h4, a4, deg, ypad, W2_self, W2_neigh, b2[None, :], codebook,
        Wd1, bd1[None, :], wd2p, bd2p[None, :])

    recon = rec16[:, :D_IN]
    indices = idxo[:, 0]
    vq_loss = (1.0 + BETA) * ssq[0, 0] / (N * D_LAT)
    recon_loss = sab[0, 0] / (N * D_IN)
    total_loss = recon_loss + vq_loss
    return recon, vq_loss, recon_loss, total_loss, indices, z_e, z_q


# speedup vs baseline: 5.3121x; 1.0383x over previous
import functools

import jax
import jax.numpy as jnp
from jax import lax
from jax.experimental import pallas as pl
from jax.experimental.pallas import tpu as pltpu
from jax.experimental.pallas import tpu_sc as plsc

N = 50000
E = 800000
D_IN = 9
D_LAT = 128
K = 1024
BETA = 0.25

NC = 2
NS = 16
NA = 50048
EROWS = 6400
EPAD = EROWS * 128 - E

_mesh = plsc.VectorSubcoreMesh(core_axis_name="c", subcore_axis_name="s",
                               num_cores=NC, num_subcores=NS)


def _seg_scratch(R):
    return [
        pltpu.VMEM_SHARED((NA, 16), jnp.float32),
        pltpu.VMEM((R * 128,), jnp.int32),
        pltpu.VMEM((R * 128,), jnp.int32),
        pltpu.VMEM((R, 128), jnp.int32),
        pltpu.VMEM((R, 128), jnp.int32),
        pltpu.VMEM((R, 128, 16), jnp.float32),
        pltpu.VMEM((R, 128, 16), jnp.float32),
        pltpu.SemaphoreType.DMA,
        pltpu.SemaphoreType.DMA,
        pltpu.SemaphoreType.DMA,
        pltpu.SemaphoreType.DMA,
    ]


def _edge_pass(table_hbm, src_hbm, dst_hbm, zeros_hbm, acc, sets,
               R, row0, npairs, tail, off):
    def load_and_gather(si, b):
        sbuf, dbuf, rows, gsem, _ = sets[si]
        r = row0 + b * R
        pltpu.sync_copy(src_hbm.at[pl.ds(r * 128, R * 128)], sbuf)
        pltpu.sync_copy(dst_hbm.at[pl.ds(r, R)], dbuf)
        if off is not None:
            offv = jnp.full((16,), off, jnp.int32)
            for i in range(R * 8):
                sl = pl.ds(i * 16, 16)
                sbuf[sl] = sbuf[sl] + offv
        for j in range(R):
            pltpu.async_copy(table_hbm.at[sbuf.at[pl.ds(j * 128, 128)]],
                             rows.at[j], gsem)

    def drain(si, which):
        _, _, rows, gsem, ssem = sets[si]
        sem = gsem if which == "g" else ssem
        for j in range(R):
            pltpu.make_async_copy(zeros_hbm.at[pl.ds(0, 128)],
                                  rows.at[j], sem).wait()

    def fire_scatter(si):
        _, dbuf, rows, _, ssem = sets[si]
        for j in range(R):
            pltpu.async_copy(rows.at[j], acc.at[dbuf.at[j]], ssem, add=True)

    load_and_gather(0, 0)

    def body(i, carry):
        @pl.when(i > 0)
        def _():
            drain(1, "s")
        load_and_gather(1, 2 * i + 1)
        drain(0, "g")
        fire_scatter(0)
        drain(0, "s")
        @pl.when(i < npairs - 1)
        def _():
            load_and_gather(0, 2 * i + 2)
        drain(1, "g")
        fire_scatter(1)
        return carry

    lax.fori_loop(0, npairs, body, 0)
    drain(1, "s")
    if tail:
        load_and_gather(0, 2 * npairs)
        drain(0, "g")
        fire_scatter(0)
        drain(0, "s")


@functools.partial(
    pl.kernel,
    out_type=jax.ShapeDtypeStruct((2 * NA, 16), jnp.float32),
    mesh=_mesh,
    compiler_params=pltpu.CompilerParams(use_tc_tiling_on_sc=False),
    scratch_types=_seg_scratch(8),
)
def _seg16(xpad_hbm, src_hbm, dst_hbm, zeros_hbm, out_hbm,
           acc, sbuf0, sbuf1, dbuf0, dbuf1, rows0, rows1,
           gsem0, gsem1, ssem0, ssem1):
    c = lax.axis_index("c")
    t = lax.axis_index("s")
    sets = ((sbuf0, dbuf0, rows0, gsem0, ssem0),
            (sbuf1, dbuf1, rows1, gsem1, ssem1))
    pltpu.sync_copy(zeros_hbm.at[pl.ds(t * (NA // NS), NA // NS)],
                    acc.at[pl.ds(t * (NA // NS), NA // NS)])
    plsc.subcore_barrier()

    row0 = c * (EROWS // 2) + t * (EROWS // 2 // NS)
    _edge_pass(xpad_hbm, src_hbm, dst_hbm, zeros_hbm, acc, sets,
               R=8, row0=row0, npairs=12, tail=True, off=None)

    plsc.subcore_barrier()
    pltpu.sync_copy(acc.at[pl.ds(t * (NA // NS), NA // NS)],
                    out_hbm.at[pl.ds(c * NA + t * (NA // NS), NA // NS)])


@functools.partial(
    pl.kernel,
    out_type=jax.ShapeDtypeStruct((4 * NA, 32), jnp.float32),
    mesh=_mesh,
    compiler_params=pltpu.CompilerParams(use_tc_tiling_on_sc=False),
    scratch_types=[
        pltpu.VMEM_SHARED((NA, 32), jnp.float32),
        pltpu.VMEM((2048,), jnp.int32),
        pltpu.VMEM((16, 128), jnp.int32),
        pltpu.VMEM((2, 128, 32), jnp.float32),
        pltpu.VMEM((2, 128, 32), jnp.float32),
        pltpu.VMEM((2, 128, 32), jnp.float32),
        pltpu.SemaphoreType.DMA,
        pltpu.SemaphoreType.DMA,
        pltpu.SemaphoreType.DMA,
        pltpu.SemaphoreType.DMA,
        pltpu.SemaphoreType.DMA,
        pltpu.SemaphoreType.DMA,
    ],
)
def _seg32(hcat_hbm, src_hbm, dst_hbm, zeros_hbm, out_hbm,
           acc, sbuf, dbuf, rowsA, rowsB, rowsC,
           gsemA, gsemB, gsemC, ssemA, ssemB, ssemC):
    c = lax.axis_index("c")
    t = lax.axis_index("s")
    rows = (rowsA, rowsB, rowsC)
    gsem = (gsemA, gsemB, gsemC)
    ssem = (ssemA, ssemB, ssemC)
    for k in range(2):
        chunk = c * 2 + k
        off = chunk * N
        pltpu.sync_copy(zeros_hbm.at[pl.ds(t * (NA // NS), NA // NS)],
                        acc.at[pl.ds(t * (NA // NS), NA // NS)])
        plsc.subcore_barrier()

        def fire_g(sub):
            bi = sub % 3
            for j in range(2):
                r = 2 * sub + j
                pltpu.async_copy(
                    hcat_hbm.at[sbuf.at[pl.ds(r * 128, 128)]],
                    rows[bi].at[j], gsem[bi])

        def fire_s(sub):
            bi = sub % 3
            for j in range(2):
                r = 2 * sub + j
                pltpu.async_copy(rows[bi].at[j], acc.at[dbuf.at[r]],
                                 ssem[bi], add=True)

        def drain(sub, sems):
            bi = sub % 3
            for j in range(2):
                pltpu.make_async_copy(zeros_hbm.at[pl.ds(0, 128)],
                                      rows[bi].at[j], sems[bi]).wait()

        def body(b, carry):
            r0 = t * (EROWS // NS) + b * 16
            pltpu.sync_copy(src_hbm.at[pl.ds(r0 * 128, 2048)], sbuf)
            pltpu.sync_copy(dst_hbm.at[pl.ds(r0, 16)], dbuf)
            offv = jnp.full((16,), off, jnp.int32)
            for i in range(128):
                sl = pl.ds(i * 16, 16)
                sbuf[sl] = sbuf[sl] + offv
            fire_g(0)
            fire_g(1)
            fire_g(2)
            for sub in range(8):
                drain(sub, gsem)
                fire_s(sub)
                if sub >= 1 and sub + 2 <= 7:
                    drain(sub - 1, ssem)
                    fire_g(sub + 2)
            for sub in range(5, 8):
                drain(sub, ssem)
            return carry

        lax.fori_loop(0, EROWS // NS // 16, body, 0)
        plsc.subcore_barrier()
        pltpu.sync_copy(acc.at[pl.ds(t * (NA // NS), NA // NS)],
                        out_hbm.at[pl.ds(chunk * NA + t * (NA // NS), NA // NS)])
        plsc.subcore_barrier()


_BN = 1000


def _enc_body(xp_ref, p_ref, w1s_ref, w1n_ref, b1_ref, h4_ref, deg_ref):
    p = p_ref[0] + p_ref[1]
    deg = jnp.maximum(p[:, 9:10], 1.0)
    agg = p / deg
    hp = (jnp.dot(xp_ref[...], w1s_ref[...], preferred_element_type=jnp.float32)
          + jnp.dot(agg, w1n_ref[...], preferred_element_type=jnp.float32)
          + b1_ref[...])
    h = jnp.maximum(hp, 0.0)
    h4_ref[...] = jnp.stack([h[:, 32 * j:32 * j + 32] for j in range(4)])
    deg_ref[...] = deg


def _enc(xpad, parts, w1s, w1n, b1):
    return pl.pallas_call(
        _enc_body,
        grid=(N // _BN,),
        in_specs=[
            pl.BlockSpec((_BN, 16), lambda i: (i, 0)),
            pl.BlockSpec((2, _BN, 16), lambda i: (0, i, 0)),
            pl.BlockSpec((16, D_LAT), lambda i: (0, 0)),
            pl.BlockSpec((16, D_LAT), lambda i: (0, 0)),
            pl.BlockSpec((1, D_LAT), lambda i: (0, 0)),
        ],
        out_specs=[
            pl.BlockSpec((4, _BN, 32), lambda i: (0, i, 0)),
            pl.BlockSpec((_BN, 1), lambda i: (i, 0)),
        ],
        out_shape=[
            jax.ShapeDtypeStruct((4, N, 32), jnp.float32),
            jax.ShapeDtypeStruct((N, 1), jnp.float32),
        ],
    )(xpad, parts, w1s, w1n, b1)


def _head_body(h4_ref, a4_ref, deg_ref, ypad_ref, w2s_ref, w2n_ref, b2_ref,
               cb_ref, wd1_ref, bd1_ref, wd2_ref, bd2_ref,
               ze_ref, zq_ref, idx_ref, rec_ref, ssq_ref, sab_ref):
    hp = lambda a, b: jnp.dot(a, b, preferred_element_type=jnp.float32)
    h = jnp.concatenate([h4_ref[j] for j in range(4)], axis=1)
    agg = jnp.concatenate([a4_ref[j] for j in range(4)], axis=1)
    agg = agg / deg_ref[...]
    z_e = hp(h, w2s_ref[...]) + hp(agg, w2n_ref[...]) + b2_ref[...]
    cb = cb_ref[...]
    d = (jnp.sum(z_e * z_e, axis=1, keepdims=True)
         + jnp.sum(cb * cb, axis=1)[None, :]
         - 2.0 * lax.dot_general(z_e, cb, (((1,), (1,)), ((), ())),
                                 preferred_element_type=jnp.float32))
    mind = jnp.min(d, axis=1, keepdims=True)
    iota = lax.broadcasted_iota(jnp.int32, d.shape, 1)
    idx = jnp.min(jnp.where(d == mind, iota, K), axis=1, keepdims=True)
    onehot = (iota == idx).astype(jnp.float32)
    z_q = hp(onehot, cb)
    hd = jnp.maximum(hp(z_q, wd1_ref[...]) + bd1_ref[...], 0.0)
    rec = hp(hd, wd2_ref[...]) + bd2_ref[...]
    ze_ref[...] = z_e
    zq_ref[...] = z_q
    idx_ref[...] = idx
    rec_ref[...] = rec

    @pl.when(pl.program_id(0) == 0)
    def _():
        ssq_ref[...] = jnp.zeros_like(ssq_ref)
        sab_ref[...] = jnp.zeros_like(sab_ref)

    dz = z_q - z_e
    ssq_ref[...] += jnp.sum(dz * dz).reshape(1, 1)
    sab_ref[...] += jnp.sum(jnp.abs(rec - ypad_ref[...])).reshape(1, 1)


def _head(h4, a4, deg, ypad, w2s, w2n, b2, cb, wd1, bd1, wd2p, bd2p):
    full = lambda *shape: pl.BlockSpec(shape, lambda i: tuple(0 for _ in shape))
    return pl.pallas_call(
        _head_body,
        grid=(N // _BN,),
        in_specs=[
            pl.BlockSpec((4, _BN, 32), lambda i: (0, i, 0)),
            pl.BlockSpec((4, _BN, 32), lambda i: (0, i, 0)),
            pl.BlockSpec((_BN, 1), lambda i: (i, 0)),
            pl.BlockSpec((_BN, 16), lambda i: (i, 0)),
            full(D_LAT, D_LAT),
            full(D_LAT, D_LAT),
            full(1, D_LAT),
            full(K, D_LAT),
            full(D_LAT, D_LAT),
            full(1, D_LAT),
            full(D_LAT, 16),
            full(1, 16),
        ],
        out_specs=[
            pl.BlockSpec((_BN, D_LAT), lambda i: (i, 0)),
            pl.BlockSpec((_BN, D_LAT), lambda i: (i, 0)),
            pl.BlockSpec((_BN, 1), lambda i: (i, 0)),
            pl.BlockSpec((_BN, 16), lambda i: (i, 0)),
            pl.BlockSpec((1, 1), lambda i: (0, 0)),
            pl.BlockSpec((1, 1), lambda i: (0, 0)),
        ],
        out_shape=[
            jax.ShapeDtypeStruct((N, D_LAT), jnp.float32),
            jax.ShapeDtypeStruct((N, D_LAT), jnp.float32),
            jax.ShapeDtypeStruct((N, 1), jnp.int32),
            jax.ShapeDtypeStruct((N, 16), jnp.float32),
            jax.ShapeDtypeStruct((1, 1), jnp.float32),
            jax.ShapeDtypeStruct((1, 1), jnp.float32),
        ],
    )(h4, a4, deg, ypad, w2s, w2n, b2, cb, wd1, bd1, wd2p, bd2p)


def kernel(x, edge_index, y, W1_self, W1_neigh, b1, W2_self, W2_neigh, b2,
           codebook, Wd1, bd1, Wd2, bd2):
    f32 = jnp.float32
    src = edge_index[0]
    dst = edge_index[1]
    src_p = jnp.concatenate([src, jnp.zeros((EPAD,), jnp.int32)])
    trash = N + (jnp.arange(EPAD, dtype=jnp.int32) % 48)
    dst2d = jnp.concatenate([dst, trash]).reshape(EROWS, 128)

    xpad = jnp.concatenate(
        [x, jnp.ones((N, 1), f32), jnp.zeros((N, 6), f32)], axis=1)
    ypad = jnp.concatenate([y, jnp.zeros((N, 7), f32)], axis=1)
    w1s = jnp.concatenate([W1_self, jnp.zeros((7, D_LAT), f32)], axis=0)
    w1n = jnp.concatenate([W1_neigh, jnp.zeros((7, D_LAT), f32)], axis=0)
    wd2p = jnp.concatenate([Wd2, jnp.zeros((D_LAT, 7), f32)], axis=1)
    bd2p = jnp.concatenate([bd2, jnp.zeros((7,), f32)])
    zeros16 = jnp.zeros((NA, 16), f32)
    zeros32 = jnp.zeros((NA, 32), f32)

    parts = _seg16(xpad, src_p, dst2d, zeros16).reshape(2, NA, 16)
    h4, deg = _enc(xpad, parts, w1s, w1n, b1[None, :])
    a4 = _seg32(h4.reshape(4 * N, 32), src_p, dst2d, zeros32).reshape(4, NA, 32)
    z_e, z_q, idxo, rec16, ssq, sab = _head(
        h4, a4, deg, ypad, W2_self, W2_neigh, b2[None, :], codebook,
        Wd1, bd1[None, :], wd2p, bd2p[None, :])

    recon = rec16[:, :D_IN]
    indices = idxo[:, 0]
    vq_loss = (1.0 + BETA) * ssq[0, 0] / (N * D_LAT)
    recon_loss = sab[0, 0] / (N * D_IN)
    total_loss = recon_loss + vq_loss
    return recon, vq_loss, recon_loss, total_loss, indices, z_e, z_q
